# baseline (speedup 1.0000x reference)
import functools

import jax
import jax.numpy as jnp
from jax import lax
from jax.experimental import pallas as pl
from jax.experimental.pallas import tpu as pltpu
from jax.experimental.pallas import tpu_sc as plsc

NUM_EMB = 1_000_000
DIM = 32
HIST = 50
BATCH = 16384

NC = 2
NS = 16
NW = NC * NS

BAGS_PER_W = BATCH // NW
CHUNK = 32
NCHUNK = BAGS_PER_W // CHUNK
ROWS = CHUNK * HIST
GROUP = 100
GPC = ROWS // GROUP
LANES = 16
SCALE = 1.0 / HIST

K1_W = 512
K1_NFULL = (NUM_EMB // K1_W)
K1_TAIL = NUM_EMB - K1_NFULL * K1_W
K1_SLOTS = -(-K1_NFULL // NW)

_mesh = plsc.VectorSubcoreMesh(
    core_axis_name="c", subcore_axis_name="s", num_cores=NC, num_subcores=NS
)


@functools.partial(
    pl.kernel,
    out_type=jax.ShapeDtypeStruct((NUM_EMB * DIM,), jnp.float32),
    mesh=_mesh,
    scratch_types=[
        pltpu.VMEM((DIM, K1_W), jnp.float32),
        pltpu.VMEM((DIM, K1_W), jnp.float32),
        pltpu.VMEM((K1_W * DIM,), jnp.float32),
        pltpu.VMEM((K1_W * DIM,), jnp.float32),
        pltpu.VMEM((K1_TAIL, DIM), jnp.float32),
        pltpu.SemaphoreType.DMA,
        pltpu.SemaphoreType.DMA,
    ],
    compiler_params=pltpu.CompilerParams(needs_layout_passes=False),
)
def _linearize(tview_hbm, tail_hbm, flat_hbm, t_v0, t_v1, g_v0, g_v1, tail_v,
               isem, osem):
    t_vs = (t_v0, t_v1)
    g_vs = (g_v0, g_v1)
    cid = lax.axis_index("c")
    sid = lax.axis_index("s")
    wid = sid * NC + cid
    iota32 = lax.iota(jnp.int32, LANES) * DIM

    def active(i):
        return wid + NW * i < K1_NFULL

    def col0(i):
        return (wid + NW * i) * K1_W

    def read(i, buf):
        pltpu.async_copy(
            tview_hbm.at[:, pl.ds(col0(i), K1_W)], t_vs[buf], isem
        )

    def wait_read(buf):
        pltpu.make_async_copy(
            tview_hbm.at[:, pl.ds(0, K1_W)], t_vs[buf], isem
        ).wait()

    def write(i, buf):
        pltpu.async_copy(
            g_vs[buf], flat_hbm.at[pl.ds(col0(i) * DIM, K1_W * DIM)], osem
        )

    def drain_write():
        pltpu.make_async_copy(
            g_vs[0], flat_hbm.at[pl.ds(0, K1_W * DIM)], osem
        ).wait()

    def transpose(buf, ngroups):
        def grp(i, carry):
            base = i * LANES
            for j in range(DIM):
                v = t_vs[buf][j, pl.ds(base, LANES)]
                plsc.store_scatter(g_vs[buf], [base * DIM + j + iota32], v)
            return carry
        lax.fori_loop(0, ngroups, grp, 0)

    read(0, 0)

    @pl.loop(0, K1_SLOTS, step=2)
    def _pair(c):
        for b in range(2):
            i = c + b

            @pl.when(active(i + 1))
            def _():
                read(i + 1, 1 - b)

            @pl.when(active(i))
            def _():
                wait_read(b)

                @pl.when(i >= 2)
                def _():
                    drain_write()

                transpose(b, K1_W // LANES)
                write(i, b)

    drain_write()
    drain_write()

    @pl.when(wid == 0)
    def _():
        pltpu.sync_copy(tail_hbm, tail_v)

        def tail_row(r, carry):
            g_v0[pl.ds(r * DIM, LANES)] = tail_v[r, pl.ds(0, LANES)]
            g_v0[pl.ds(r * DIM + LANES, LANES)] = tail_v[r, pl.ds(LANES, LANES)]
            return carry

        lax.fori_loop(0, K1_TAIL, tail_row, 0)
        pltpu.sync_copy(
            g_v0.at[pl.ds(0, K1_TAIL * DIM)],
            flat_hbm.at[pl.ds(K1_NFULL * K1_W * DIM, K1_TAIL * DIM)],
        )


@functools.partial(
    pl.kernel,
    out_type=jax.ShapeDtypeStruct((BATCH, DIM), jnp.float32),
    mesh=_mesh,
    scratch_types=[
        pltpu.VMEM((2, GPC, GROUP), jnp.int32),
        pltpu.VMEM((2, ROWS, DIM), jnp.float32),
        pltpu.VMEM((CHUNK, DIM), jnp.float32),
        pltpu.SemaphoreType.DMA,
        pltpu.SemaphoreType.DMA,
    ],
    compiler_params=pltpu.CompilerParams(use_tc_tiling_on_sc=False),
)
def _embbag(idx_hbm, table_hbm, out_hbm, idx_v, rows_v, out_v, sem0, sem1):
    cid = lax.axis_index("c")
    sid = lax.axis_index("s")
    wid = sid * NC + cid
    gbase = wid * (BAGS_PER_W * HIST // GROUP)
    bagbase = wid * BAGS_PER_W
    sems = (sem0, sem1)

    def stage(c, buf):
        pltpu.sync_copy(idx_hbm.at[pl.ds(gbase + c * GPC, GPC)], idx_v.at[buf])
        for g in range(GPC):
            pltpu.async_copy(
                table_hbm.at[idx_v.at[buf, g]],
                rows_v.at[buf, pl.ds(g * GROUP, GROUP)],
                sems[buf],
            )

    def drain(buf):
        pltpu.make_async_copy(
            table_hbm.at[pl.ds(0, ROWS)], rows_v.at[buf], sems[buf]
        ).wait()

    def compute(c, buf):
        def bag_body(b, carry):
            r0 = b * HIST
            acc0 = rows_v[buf, r0, pl.ds(0, LANES)]
            acc1 = rows_v[buf, r0, pl.ds(LANES, LANES)]
            for j in range(1, HIST):
                acc0 = acc0 + rows_v[buf, r0 + j, pl.ds(0, LANES)]
                acc1 = acc1 + rows_v[buf, r0 + j, pl.ds(LANES, LANES)]
            out_v[b, pl.ds(0, LANES)] = acc0 * SCALE
            out_v[b, pl.ds(LANES, LANES)] = acc1 * SCALE
            return carry
        lax.fori_loop(0, CHUNK, bag_body, 0)
        pltpu.sync_copy(out_v, out_hbm.at[pl.ds(bagbase + c * CHUNK, CHUNK)])

    stage(0, 0)

    @pl.loop(0, NCHUNK, step=2)
    def _chunk_pair(c):
        for buf in range(2):
            cc = c + buf

            @pl.when(cc + 1 < NCHUNK)
            def _():
                stage(cc + 1, 1 - buf)

            drain(buf)
            compute(cc, buf)


def kernel(input, table):
    idx = input.astype(jnp.int32).reshape(BATCH * HIST // GROUP, GROUP)
    tail = table[NUM_EMB - K1_TAIL:, :]
    table_lin = _linearize(table.T, tail).reshape(NUM_EMB, DIM)
    return _embbag(idx, table_lin)


# speedup vs baseline: 1.1942x; 1.1942x over previous
import functools

import jax
import jax.numpy as jnp
from jax import lax
from jax.experimental import pallas as pl
from jax.experimental.pallas import tpu as pltpu
from jax.experimental.pallas import tpu_sc as plsc

NUM_EMB = 1_000_000
DIM = 32
HIST = 50
BATCH = 16384

NC = 2
NS = 16
NW = NC * NS

BAGS_PER_W = BATCH // NW
CHUNK = 32
NCHUNK = BAGS_PER_W // CHUNK
ROWS = CHUNK * HIST
GROUP = 100
GPC = ROWS // GROUP
LANES = 16
SCALE = 1.0 / HIST

K1_W = 512
K1_NFULL = (NUM_EMB // K1_W)
K1_TAIL = NUM_EMB - K1_NFULL * K1_W
K1_SLOTS = -(-K1_NFULL // NW)

_mesh = plsc.VectorSubcoreMesh(
    core_axis_name="c", subcore_axis_name="s", num_cores=NC, num_subcores=NS
)


@functools.partial(
    pl.kernel,
    out_type=jax.ShapeDtypeStruct((NUM_EMB * DIM,), jnp.float32),
    mesh=_mesh,
    scratch_types=[
        pltpu.VMEM((DIM, K1_W), jnp.float32),
        pltpu.VMEM((DIM, K1_W), jnp.float32),
        pltpu.VMEM((K1_W * DIM,), jnp.float32),
        pltpu.VMEM((K1_W * DIM,), jnp.float32),
        pltpu.VMEM((K1_TAIL, DIM), jnp.float32),
        pltpu.SemaphoreType.DMA,
        pltpu.SemaphoreType.DMA,
    ],
    compiler_params=pltpu.CompilerParams(needs_layout_passes=False),
)
def _linearize(tview_hbm, tail_hbm, flat_hbm, t_v0, t_v1, g_v0, g_v1, tail_v,
               isem, osem):
    t_vs = (t_v0, t_v1)
    g_vs = (g_v0, g_v1)
    cid = lax.axis_index("c")
    sid = lax.axis_index("s")
    wid = sid * NC + cid
    iota32 = lax.iota(jnp.int32, LANES) * DIM

    def active(i):
        return wid + NW * i < K1_NFULL

    def col0(i):
        return (wid + NW * i) * K1_W

    def read(i, buf):
        pltpu.async_copy(
            tview_hbm.at[:, pl.ds(col0(i), K1_W)], t_vs[buf], isem
        )

    def wait_read(buf):
        pltpu.make_async_copy(
            tview_hbm.at[:, pl.ds(0, K1_W)], t_vs[buf], isem
        ).wait()

    def write(i, buf):
        pltpu.async_copy(
            g_vs[buf], flat_hbm.at[pl.ds(col0(i) * DIM, K1_W * DIM)], osem
        )

    def drain_write():
        pltpu.make_async_copy(
            g_vs[0], flat_hbm.at[pl.ds(0, K1_W * DIM)], osem
        ).wait()

    def transpose(buf, ngroups):
        @plsc.parallel_loop(0, ngroups, unroll=2)
        def grp(i):
            base = i * LANES
            for j in range(DIM):
                v = t_vs[buf][j, pl.ds(base, LANES)]
                plsc.store_scatter(g_vs[buf], [base * DIM + j + iota32], v)

    read(0, 0)

    @pl.loop(0, K1_SLOTS, step=2)
    def _pair(c):
        for b in range(2):
            i = c + b

            @pl.when(active(i + 1))
            def _():
                read(i + 1, 1 - b)

            @pl.when(active(i))
            def _():
                wait_read(b)

                @pl.when(i >= 2)
                def _():
                    drain_write()

                transpose(b, K1_W // LANES)
                write(i, b)

    drain_write()
    drain_write()

    @pl.when(wid == 0)
    def _():
        pltpu.sync_copy(tail_hbm, tail_v)

        def tail_row(r, carry):
            g_v0[pl.ds(r * DIM, LANES)] = tail_v[r, pl.ds(0, LANES)]
            g_v0[pl.ds(r * DIM + LANES, LANES)] = tail_v[r, pl.ds(LANES, LANES)]
            return carry

        lax.fori_loop(0, K1_TAIL, tail_row, 0)
        pltpu.sync_copy(
            g_v0.at[pl.ds(0, K1_TAIL * DIM)],
            flat_hbm.at[pl.ds(K1_NFULL * K1_W * DIM, K1_TAIL * DIM)],
        )


@functools.partial(
    pl.kernel,
    out_type=jax.ShapeDtypeStruct((BATCH, DIM), jnp.float32),
    mesh=_mesh,
    scratch_types=[
        pltpu.VMEM((2, GPC, GROUP), jnp.int32),
        pltpu.VMEM((2, ROWS, DIM), jnp.float32),
        pltpu.VMEM((CHUNK, DIM), jnp.float32),
        pltpu.SemaphoreType.DMA,
        pltpu.SemaphoreType.DMA,
    ],
    compiler_params=pltpu.CompilerParams(use_tc_tiling_on_sc=False),
)
def _embbag(idx_hbm, table_hbm, out_hbm, idx_v, rows_v, out_v, sem0, sem1):
    cid = lax.axis_index("c")
    sid = lax.axis_index("s")
    wid = sid * NC + cid
    gbase = wid * (BAGS_PER_W * HIST // GROUP)
    bagbase = wid * BAGS_PER_W
    sems = (sem0, sem1)

    def stage(c, buf):
        pltpu.sync_copy(idx_hbm.at[pl.ds(gbase + c * GPC, GPC)], idx_v.at[buf])
        for g in range(GPC):
            pltpu.async_copy(
                table_hbm.at[idx_v.at[buf, g]],
                rows_v.at[buf, pl.ds(g * GROUP, GROUP)],
                sems[buf],
            )

    def drain(buf):
        pltpu.make_async_copy(
            table_hbm.at[pl.ds(0, ROWS)], rows_v.at[buf], sems[buf]
        ).wait()

    def compute(c, buf):
        @plsc.parallel_loop(0, CHUNK, unroll=2)
        def bag_body(b):
            r0 = b * HIST
            acc0 = rows_v[buf, r0, pl.ds(0, LANES)]
            acc1 = rows_v[buf, r0, pl.ds(LANES, LANES)]
            for j in range(1, HIST):
                acc0 = acc0 + rows_v[buf, r0 + j, pl.ds(0, LANES)]
                acc1 = acc1 + rows_v[buf, r0 + j, pl.ds(LANES, LANES)]
            out_v[b, pl.ds(0, LANES)] = acc0 * SCALE
            out_v[b, pl.ds(LANES, LANES)] = acc1 * SCALE
        pltpu.sync_copy(out_v, out_hbm.at[pl.ds(bagbase + c * CHUNK, CHUNK)])

    stage(0, 0)

    @pl.loop(0, NCHUNK, step=2)
    def _chunk_pair(c):
        for buf in range(2):
            cc = c + buf

            @pl.when(cc + 1 < NCHUNK)
            def _():
                stage(cc + 1, 1 - buf)

            drain(buf)
            compute(cc, buf)


def kernel(input, table):
    idx = input.astype(jnp.int32).reshape(BATCH * HIST // GROUP, GROUP)
    tail = table[NUM_EMB - K1_TAIL:, :]
    table_lin = _linearize(table.T, tail).reshape(NUM_EMB, DIM)
    return _embbag(idx, table_lin)


# speedup vs baseline: 3.3381x; 2.7952x over previous
import functools

import jax
import jax.numpy as jnp
from jax import lax
from jax.experimental import pallas as pl
from jax.experimental.pallas import tpu as pltpu
from jax.experimental.pallas import tpu_sc as plsc

NUM_EMB = 1_000_000
DIM = 32
HIST = 50
BATCH = 16384

NC = 2
NS = 16
NW = NC * NS

BAGS_PER_W = BATCH // NW
CHUNK = 32
NCHUNK = BAGS_PER_W // CHUNK
ROWS = CHUNK * HIST
GROUP = 100
GPC = ROWS // GROUP
LANES = 16
SCALE = 1.0 / HIST

K1_W = 512
K1_NFULL = NUM_EMB // K1_W
K1_TAIL = NUM_EMB - K1_NFULL * K1_W
K1_SLOTS = -(-K1_NFULL // NW)
PPITCH = DIM + 1

_mesh = plsc.VectorSubcoreMesh(
    core_axis_name="c", subcore_axis_name="s", num_cores=NC, num_subcores=NS
)


@functools.partial(
    pl.kernel,
    out_type=jax.ShapeDtypeStruct((NUM_EMB * DIM,), jnp.float32),
    mesh=_mesh,
    scratch_types=[
        pltpu.VMEM((DIM, K1_W), jnp.float32),
        pltpu.VMEM((DIM, K1_W), jnp.float32),
        pltpu.VMEM((K1_W * DIM,), jnp.float32),
        pltpu.VMEM((K1_W * DIM,), jnp.float32),
        pltpu.VMEM((K1_W * PPITCH,), jnp.float32),
        pltpu.VMEM((K1_TAIL, DIM), jnp.float32),
        pltpu.SemaphoreType.DMA,
        pltpu.SemaphoreType.DMA,
    ],
    compiler_params=pltpu.CompilerParams(needs_layout_passes=False),
)
def _linearize(tview_hbm, tail_hbm, flat_hbm, t_v0, t_v1, g_v0, g_v1, p_v,
               tail_v, isem, osem):
    t_vs = (t_v0, t_v1)
    g_vs = (g_v0, g_v1)
    cid = lax.axis_index("c")
    sid = lax.axis_index("s")
    wid = sid * NC + cid
    iota1 = lax.iota(jnp.int32, LANES)
    iota_p = iota1 * PPITCH

    def active(i):
        return wid + NW * i < K1_NFULL

    def col0(i):
        return (wid + NW * i) * K1_W

    def read(i, buf):
        pltpu.async_copy(
            tview_hbm.at[:, pl.ds(col0(i), K1_W)], t_vs[buf], isem
        )

    def wait_read(buf):
        pltpu.make_async_copy(
            tview_hbm.at[:, pl.ds(0, K1_W)], t_vs[buf], isem
        ).wait()

    def write(i, buf):
        pltpu.async_copy(
            g_vs[buf], flat_hbm.at[pl.ds(col0(i) * DIM, K1_W * DIM)], osem
        )

    def drain_write():
        pltpu.make_async_copy(
            g_vs[0], flat_hbm.at[pl.ds(0, K1_W * DIM)], osem
        ).wait()

    def transpose(buf):
        @plsc.parallel_loop(0, K1_W // LANES, unroll=2)
        def grp(i):
            base = i * LANES
            for j in range(DIM):
                v = t_vs[buf][j, pl.ds(base, LANES)]
                plsc.store_scatter(p_v, [base * PPITCH + j + iota_p], v)

        @plsc.parallel_loop(0, K1_W, unroll=4)
        def row(r):
            g_vs[buf][pl.ds(r * DIM, LANES)] = plsc.load_gather(
                p_v, [r * PPITCH + iota1]
            )
            g_vs[buf][pl.ds(r * DIM + LANES, LANES)] = plsc.load_gather(
                p_v, [r * PPITCH + LANES + iota1]
            )

    read(0, 0)

    @pl.loop(0, K1_SLOTS, step=2)
    def _pair(c):
        for b in range(2):
            i = c + b

            @pl.when(active(i + 1))
            def _():
                read(i + 1, 1 - b)

            @pl.when(active(i))
            def _():
                wait_read(b)

                @pl.when(i >= 2)
                def _():
                    drain_write()

                transpose(b)
                write(i, b)

    drain_write()
    drain_write()

    @pl.when(wid == 0)
    def _():
        pltpu.sync_copy(tail_hbm, tail_v)

        @plsc.parallel_loop(0, K1_TAIL, unroll=4)
        def tail_row(r):
            g_v0[pl.ds(r * DIM, LANES)] = tail_v[r, pl.ds(0, LANES)]
            g_v0[pl.ds(r * DIM + LANES, LANES)] = tail_v[r, pl.ds(LANES, LANES)]

        pltpu.sync_copy(
            g_v0.at[pl.ds(0, K1_TAIL * DIM)],
            flat_hbm.at[pl.ds(K1_NFULL * K1_W * DIM, K1_TAIL * DIM)],
        )


@functools.partial(
    pl.kernel,
    out_type=jax.ShapeDtypeStruct((BATCH, DIM), jnp.float32),
    mesh=_mesh,
    scratch_types=[
        pltpu.VMEM((2, GPC, GROUP), jnp.int32),
        pltpu.VMEM((2, ROWS, DIM), jnp.float32),
        pltpu.VMEM((CHUNK, DIM), jnp.float32),
        pltpu.SemaphoreType.DMA,
        pltpu.SemaphoreType.DMA,
    ],
    compiler_params=pltpu.CompilerParams(use_tc_tiling_on_sc=False),
)
def _embbag(idx_hbm, table_hbm, out_hbm, idx_v, rows_v, out_v, sem0, sem1):
    cid = lax.axis_index("c")
    sid = lax.axis_index("s")
    wid = sid * NC + cid
    gbase = wid * (BAGS_PER_W * HIST // GROUP)
    bagbase = wid * BAGS_PER_W
    sems = (sem0, sem1)

    def stage(c, buf):
        pltpu.sync_copy(idx_hbm.at[pl.ds(gbase + c * GPC, GPC)], idx_v.at[buf])
        for g in range(GPC):
            pltpu.async_copy(
                table_hbm.at[idx_v.at[buf, g]],
                rows_v.at[buf, pl.ds(g * GROUP, GROUP)],
                sems[buf],
            )

    def drain(buf):
        pltpu.make_async_copy(
            table_hbm.at[pl.ds(0, ROWS)], rows_v.at[buf], sems[buf]
        ).wait()

    def compute(c, buf):
        @plsc.parallel_loop(0, CHUNK, unroll=2)
        def bag_body(b):
            r0 = b * HIST
            acc0 = rows_v[buf, r0, pl.ds(0, LANES)]
            acc1 = rows_v[buf, r0, pl.ds(LANES, LANES)]
            for j in range(1, HIST):
                acc0 = acc0 + rows_v[buf, r0 + j, pl.ds(0, LANES)]
                acc1 = acc1 + rows_v[buf, r0 + j, pl.ds(LANES, LANES)]
            out_v[b, pl.ds(0, LANES)] = acc0 * SCALE
            out_v[b, pl.ds(LANES, LANES)] = acc1 * SCALE
        pltpu.sync_copy(out_v, out_hbm.at[pl.ds(bagbase + c * CHUNK, CHUNK)])

    stage(0, 0)

    @pl.loop(0, NCHUNK, step=2)
    def _chunk_pair(c):
        for buf in range(2):
            cc = c + buf

            @pl.when(cc + 1 < NCHUNK)
            def _():
                stage(cc + 1, 1 - buf)

            drain(buf)
            compute(cc, buf)


def kernel(input, table):
    idx = input.astype(jnp.int32).reshape(BATCH * HIST // GROUP, GROUP)
    tail = table[NUM_EMB - K1_TAIL:, :]
    table_lin = _linearize(table.T, tail).reshape(NUM_EMB, DIM)
    return _embbag(idx, table_lin)


# speedup vs baseline: 3.4530x; 1.0344x over previous
import functools

import jax
import jax.numpy as jnp
from jax import lax
from jax.experimental import pallas as pl
from jax.experimental.pallas import tpu as pltpu
from jax.experimental.pallas import tpu_sc as plsc

NUM_EMB = 1_000_000
DIM = 32
HIST = 50
BATCH = 16384

NC = 2
NS = 16
NW = NC * NS

BAGS_PER_W = BATCH // NW
CHUNK = 32
NCHUNK = BAGS_PER_W // CHUNK
ROWS = CHUNK * HIST
GROUP = 100
GPC = ROWS // GROUP
LANES = 16
SCALE = 1.0 / HIST

K1_W = 512
K1_NFULL = NUM_EMB // K1_W
K1_TAIL = NUM_EMB - K1_NFULL * K1_W
K1_SLOTS = -(-K1_NFULL // NW)
PPITCH = DIM + 1

_mesh = plsc.VectorSubcoreMesh(
    core_axis_name="c", subcore_axis_name="s", num_cores=NC, num_subcores=NS
)


@functools.partial(
    pl.kernel,
    out_type=jax.ShapeDtypeStruct((NUM_EMB * DIM,), jnp.float32),
    mesh=_mesh,
    scratch_types=[
        pltpu.VMEM((DIM, K1_W), jnp.float32),
        pltpu.VMEM((DIM, K1_W), jnp.float32),
        pltpu.VMEM((K1_W * DIM,), jnp.float32),
        pltpu.VMEM((K1_W * DIM,), jnp.float32),
        pltpu.VMEM((K1_W * PPITCH,), jnp.float32),
        pltpu.VMEM((K1_TAIL, DIM), jnp.float32),
        pltpu.SemaphoreType.DMA,
        pltpu.SemaphoreType.DMA,
    ],
    compiler_params=pltpu.CompilerParams(needs_layout_passes=False),
)
def _linearize(tview_hbm, tail_hbm, flat_hbm, t_v0, t_v1, g_v0, g_v1, p_v,
               tail_v, isem, osem):
    t_vs = (t_v0, t_v1)
    g_vs = (g_v0, g_v1)
    cid = lax.axis_index("c")
    sid = lax.axis_index("s")
    wid = sid * NC + cid
    iota1 = lax.iota(jnp.int32, LANES)
    iota_p = iota1 * PPITCH

    def active(i):
        return wid + NW * i < K1_NFULL

    def col0(i):
        return (wid + NW * i) * K1_W

    def read(i, buf):
        pltpu.async_copy(
            tview_hbm.at[:, pl.ds(col0(i), K1_W)], t_vs[buf], isem
        )

    def wait_read(buf):
        pltpu.make_async_copy(
            tview_hbm.at[:, pl.ds(0, K1_W)], t_vs[buf], isem
        ).wait()

    def write(i, buf):
        pltpu.async_copy(
            g_vs[buf], flat_hbm.at[pl.ds(col0(i) * DIM, K1_W * DIM)], osem
        )

    def drain_write():
        pltpu.make_async_copy(
            g_vs[0], flat_hbm.at[pl.ds(0, K1_W * DIM)], osem
        ).wait()

    def transpose(buf):
        @plsc.parallel_loop(0, K1_W // LANES, unroll=4)
        def grp(i):
            base = i * LANES
            for j in range(DIM):
                v = t_vs[buf][j, pl.ds(base, LANES)]
                plsc.store_scatter(p_v, [base * PPITCH + j + iota_p], v)

        @plsc.parallel_loop(0, K1_W, unroll=4)
        def row(r):
            g_vs[buf][pl.ds(r * DIM, LANES)] = plsc.load_gather(
                p_v, [r * PPITCH + iota1]
            )
            g_vs[buf][pl.ds(r * DIM + LANES, LANES)] = plsc.load_gather(
                p_v, [r * PPITCH + LANES + iota1]
            )

    read(0, 0)

    @pl.loop(0, K1_SLOTS, step=2)
    def _pair(c):
        for b in range(2):
            i = c + b

            @pl.when(active(i + 1))
            def _():
                read(i + 1, 1 - b)

            @pl.when(active(i))
            def _():
                wait_read(b)

                @pl.when(i >= 2)
                def _():
                    drain_write()

                transpose(b)
                write(i, b)

    drain_write()
    drain_write()

    @pl.when(wid == 0)
    def _():
        pltpu.sync_copy(tail_hbm, tail_v)

        @plsc.parallel_loop(0, K1_TAIL, unroll=4)
        def tail_row(r):
            g_v0[pl.ds(r * DIM, LANES)] = tail_v[r, pl.ds(0, LANES)]
            g_v0[pl.ds(r * DIM + LANES, LANES)] = tail_v[r, pl.ds(LANES, LANES)]

        pltpu.sync_copy(
            g_v0.at[pl.ds(0, K1_TAIL * DIM)],
            flat_hbm.at[pl.ds(K1_NFULL * K1_W * DIM, K1_TAIL * DIM)],
        )


@functools.partial(
    pl.kernel,
    out_type=jax.ShapeDtypeStruct((BATCH, DIM), jnp.float32),
    mesh=_mesh,
    scratch_types=[
        pltpu.VMEM((2, GPC, GROUP), jnp.int32),
        pltpu.VMEM((2, ROWS, DIM), jnp.float32),
        pltpu.VMEM((CHUNK, DIM), jnp.float32),
        pltpu.SemaphoreType.DMA,
        pltpu.SemaphoreType.DMA,
    ],
    compiler_params=pltpu.CompilerParams(use_tc_tiling_on_sc=False),
)
def _embbag(idx_hbm, table_hbm, out_hbm, idx_v, rows_v, out_v, sem0, sem1):
    cid = lax.axis_index("c")
    sid = lax.axis_index("s")
    wid = sid * NC + cid
    gbase = wid * (BAGS_PER_W * HIST // GROUP)
    bagbase = wid * BAGS_PER_W
    sems = (sem0, sem1)

    def stage(c, buf):
        pltpu.sync_copy(idx_hbm.at[pl.ds(gbase + c * GPC, GPC)], idx_v.at[buf])
        for g in range(GPC):
            pltpu.async_copy(
                table_hbm.at[idx_v.at[buf, g]],
                rows_v.at[buf, pl.ds(g * GROUP, GROUP)],
                sems[buf],
            )

    def drain(buf):
        pltpu.make_async_copy(
            table_hbm.at[pl.ds(0, ROWS)], rows_v.at[buf], sems[buf]
        ).wait()

    def compute(c, buf):
        def bag_body(b, carry):
            r0 = b * HIST
            acc0 = rows_v[buf, r0, pl.ds(0, LANES)]
            acc1 = rows_v[buf, r0, pl.ds(LANES, LANES)]
            for j in range(1, HIST):
                acc0 = acc0 + rows_v[buf, r0 + j, pl.ds(0, LANES)]
                acc1 = acc1 + rows_v[buf, r0 + j, pl.ds(LANES, LANES)]
            out_v[b, pl.ds(0, LANES)] = acc0 * SCALE
            out_v[b, pl.ds(LANES, LANES)] = acc1 * SCALE
            return carry
        lax.fori_loop(0, CHUNK, bag_body, 0)
        pltpu.sync_copy(out_v, out_hbm.at[pl.ds(bagbase + c * CHUNK, CHUNK)])

    stage(0, 0)

    @pl.loop(0, NCHUNK, step=2)
    def _chunk_pair(c):
        for buf in range(2):
            cc = c + buf

            @pl.when(cc + 1 < NCHUNK)
            def _():
                stage(cc + 1, 1 - buf)

            drain(buf)
            compute(cc, buf)


def kernel(input, table):
    idx = input.astype(jnp.int32).reshape(BATCH * HIST // GROUP, GROUP)
    tail = table[NUM_EMB - K1_TAIL:, :]
    table_lin = _linearize(table.T, tail).reshape(NUM_EMB, DIM)
    return _embbag(idx, table_lin)


# speedup vs baseline: 3.5540x; 1.0292x over previous
import functools

import jax
import jax.numpy as jnp
from jax import lax
from jax.experimental import pallas as pl
from jax.experimental.pallas import tpu as pltpu
from jax.experimental.pallas import tpu_sc as plsc

NUM_EMB = 1_000_000
DIM = 32
HIST = 50
BATCH = 16384

NC = 2
NS = 16
NW = NC * NS

BAGS_PER_W = BATCH // NW
CHUNK = 32
NCHUNK = BAGS_PER_W // CHUNK
ROWS = CHUNK * HIST
GROUP = 100
GPC = ROWS // GROUP
LANES = 16
SCALE = 1.0 / HIST

K1_W = 512
K1_NFULL = NUM_EMB // K1_W
K1_TAIL = NUM_EMB - K1_NFULL * K1_W
K1_SLOTS = -(-K1_NFULL // NW)
PPITCH = DIM + 1

_mesh = plsc.VectorSubcoreMesh(
    core_axis_name="c", subcore_axis_name="s", num_cores=NC, num_subcores=NS
)


@functools.partial(
    pl.kernel,
    out_type=jax.ShapeDtypeStruct((NUM_EMB * DIM,), jnp.float32),
    mesh=_mesh,
    scratch_types=[
        pltpu.VMEM((DIM, K1_W), jnp.float32),
        pltpu.VMEM((DIM, K1_W), jnp.float32),
        pltpu.VMEM((K1_W * DIM,), jnp.float32),
        pltpu.VMEM((K1_W * DIM,), jnp.float32),
        pltpu.VMEM((K1_W * PPITCH,), jnp.float32),
        pltpu.VMEM((K1_TAIL, DIM), jnp.float32),
        pltpu.SemaphoreType.DMA,
        pltpu.SemaphoreType.DMA,
    ],
    compiler_params=pltpu.CompilerParams(needs_layout_passes=False),
)
def _linearize(tview_hbm, tail_hbm, flat_hbm, t_v0, t_v1, g_v0, g_v1, p_v,
               tail_v, isem, osem):
    t_vs = (t_v0, t_v1)
    g_vs = (g_v0, g_v1)
    cid = lax.axis_index("c")
    sid = lax.axis_index("s")
    wid = sid * NC + cid
    iota1 = lax.iota(jnp.int32, LANES)
    iota_p = iota1 * PPITCH

    def active(i):
        return wid + NW * i < K1_NFULL

    def col0(i):
        return (wid + NW * i) * K1_W

    def read(i, buf):
        pltpu.async_copy(
            tview_hbm.at[:, pl.ds(col0(i), K1_W)], t_vs[buf], isem
        )

    def wait_read(buf):
        pltpu.make_async_copy(
            tview_hbm.at[:, pl.ds(0, K1_W)], t_vs[buf], isem
        ).wait()

    def write(i, buf):
        pltpu.async_copy(
            g_vs[buf], flat_hbm.at[pl.ds(col0(i) * DIM, K1_W * DIM)], osem
        )

    def drain_write():
        pltpu.make_async_copy(
            g_vs[0], flat_hbm.at[pl.ds(0, K1_W * DIM)], osem
        ).wait()

    def transpose(buf):
        @plsc.parallel_loop(0, K1_W // LANES, unroll=4)
        def grp(i):
            base = i * LANES
            for j in range(DIM):
                v = t_vs[buf][j, pl.ds(base, LANES)]
                plsc.store_scatter(p_v, [base * PPITCH + j + iota_p], v)

        @plsc.parallel_loop(0, K1_W, unroll=4)
        def row(r):
            g_vs[buf][pl.ds(r * DIM, LANES)] = plsc.load_gather(
                p_v, [r * PPITCH + iota1]
            )
            g_vs[buf][pl.ds(r * DIM + LANES, LANES)] = plsc.load_gather(
                p_v, [r * PPITCH + LANES + iota1]
            )

    read(0, 0)

    @pl.loop(0, K1_SLOTS, step=2)
    def _pair(c):
        for b in range(2):
            i = c + b

            @pl.when(active(i + 1))
            def _():
                read(i + 1, 1 - b)

            @pl.when(active(i))
            def _():
                wait_read(b)

                @pl.when(i >= 2)
                def _():
                    drain_write()

                transpose(b)
                write(i, b)

    drain_write()
    drain_write()

    @pl.when(wid == 0)
    def _():
        pltpu.sync_copy(tail_hbm, tail_v)

        @plsc.parallel_loop(0, K1_TAIL, unroll=4)
        def tail_row(r):
            g_v0[pl.ds(r * DIM, LANES)] = tail_v[r, pl.ds(0, LANES)]
            g_v0[pl.ds(r * DIM + LANES, LANES)] = tail_v[r, pl.ds(LANES, LANES)]

        pltpu.sync_copy(
            g_v0.at[pl.ds(0, K1_TAIL * DIM)],
            flat_hbm.at[pl.ds(K1_NFULL * K1_W * DIM, K1_TAIL * DIM)],
        )


@functools.partial(
    pl.kernel,
    out_type=jax.ShapeDtypeStruct((BATCH, DIM), jnp.float32),
    mesh=_mesh,
    scratch_types=[
        pltpu.VMEM((2, GPC, GROUP), jnp.int32),
        pltpu.VMEM((2, ROWS, DIM), jnp.float32),
        pltpu.VMEM((CHUNK, DIM), jnp.float32),
        pltpu.SemaphoreType.DMA,
        pltpu.SemaphoreType.DMA,
        pltpu.SemaphoreType.DMA,
    ],
    compiler_params=pltpu.CompilerParams(use_tc_tiling_on_sc=False),
)
def _embbag(idx_hbm, table_hbm, out_hbm, idx_v, rows_v, out_v, sem0, sem1,
            xsem):
    cid = lax.axis_index("c")
    sid = lax.axis_index("s")
    wid = sid * NC + cid
    gbase = wid * (BAGS_PER_W * HIST // GROUP)
    bagbase = wid * BAGS_PER_W
    sems = (sem0, sem1)

    def stage_idx(c, buf):
        pltpu.async_copy(
            idx_hbm.at[pl.ds(gbase + c * GPC, GPC)], idx_v.at[buf], xsem
        )

    def wait_idx(buf):
        pltpu.make_async_copy(
            idx_hbm.at[pl.ds(0, GPC)], idx_v.at[buf], xsem
        ).wait()

    def fire(c, buf):
        for g in range(GPC):
            pltpu.async_copy(
                table_hbm.at[idx_v.at[buf, g]],
                rows_v.at[buf, pl.ds(g * GROUP, GROUP)],
                sems[buf],
            )

    def drain(buf):
        pltpu.make_async_copy(
            table_hbm.at[pl.ds(0, ROWS)], rows_v.at[buf], sems[buf]
        ).wait()

    def compute(c, buf):
        def bag_body(b, carry):
            r0 = b * HIST
            acc0 = rows_v[buf, r0, pl.ds(0, LANES)]
            acc1 = rows_v[buf, r0, pl.ds(LANES, LANES)]
            for j in range(1, HIST):
                acc0 = acc0 + rows_v[buf, r0 + j, pl.ds(0, LANES)]
                acc1 = acc1 + rows_v[buf, r0 + j, pl.ds(LANES, LANES)]
            out_v[b, pl.ds(0, LANES)] = acc0 * SCALE
            out_v[b, pl.ds(LANES, LANES)] = acc1 * SCALE
            return carry
        lax.fori_loop(0, CHUNK, bag_body, 0)
        pltpu.sync_copy(out_v, out_hbm.at[pl.ds(bagbase + c * CHUNK, CHUNK)])

    stage_idx(0, 0)
    wait_idx(0)
    fire(0, 0)
    stage_idx(1, 1)

    @pl.loop(0, NCHUNK, step=2)
    def _chunk_pair(c):
        for buf in range(2):
            cc = c + buf

            @pl.when(cc + 1 < NCHUNK)
            def _():
                wait_idx(1 - buf)
                fire(cc + 1, 1 - buf)

            drain(buf)

            @pl.when(cc + 2 < NCHUNK)
            def _():
                stage_idx(cc + 2, buf)

            compute(cc, buf)


def kernel(input, table):
    idx = input.astype(jnp.int32).reshape(BATCH * HIST // GROUP, GROUP)
    tail = table[NUM_EMB - K1_TAIL:, :]
    table_lin = _linearize(table.T, tail).reshape(NUM_EMB, DIM)
    return _embbag(idx, table_lin)
